# baseline (speedup 1.0000x reference)
import functools

import jax
import jax.numpy as jnp
from jax import lax
from jax.experimental import pallas as pl
from jax.experimental.pallas import tpu as pltpu
from jax.experimental.pallas import tpu_sc as plsc

M = 8192
F = 4096
H = 128
B = 32
MB = 512
NBLK = M // MB


HALF = MB // 2


def _argmin_body(mem_a_ref, mem_b_ref, wT_ref, W_ref, flatT_ref, thr_row_ref,
                 thr_col_ref, out_ref, gT_ref, min_ref):
    i = pl.program_id(0)

    @pl.when(i == 0)
    def _init():
        hfT_logits = jnp.dot(W_ref[...], flatT_ref[...],
                             preferred_element_type=jnp.float32)
        hfT = (hfT_logits > thr_col_ref[...]).astype(jnp.float32)
        gT_ref[...] = 1.0 - 2.0 * hfT
        min_ref[...] = jnp.full((1, B), 1e30, jnp.float32)

    wT = wT_ref[...].astype(jnp.bfloat16)
    base = jnp.float32(i * MB)
    for half, ref in ((0, mem_a_ref), (1, mem_b_ref)):
        logits = jnp.dot(ref[...].astype(jnp.bfloat16), wT,
                         preferred_element_type=jnp.float32)
        hm = (logits > thr_row_ref[...]).astype(jnp.float32)
        sT = jnp.dot(hm, gT_ref[...], preferred_element_type=jnp.float32)
        m_ids = lax.broadcasted_iota(jnp.int32, (HALF, B), 0)
        key = sT * float(M) + (m_ids.astype(jnp.float32)
                               + (base + jnp.float32(half * HALF)))
        min_ref[...] = jnp.minimum(min_ref[...],
                                   jnp.min(key, axis=0, keepdims=True))

    @pl.when(i == NBLK - 1)
    def _fin():
        k = min_ref[...]
        idx = k - jnp.floor(k / float(M)) * float(M)
        out_ref[...] = idx.astype(jnp.int32)


def _hash_argmin(memory, flat, hash_W, hash_b):
    wT = hash_W.T
    flatT = flat.T
    thr = 0.5 - hash_b
    thr_row = thr[None, :]
    thr_col = jnp.broadcast_to(thr[:, None], (H, B))

    idx2d = pl.pallas_call(
        _argmin_body,
        grid=(NBLK,),
        in_specs=[
            pl.BlockSpec((HALF, F), lambda i: (2 * i, 0)),
            pl.BlockSpec((HALF, F), lambda i: (2 * i + 1, 0)),
            pl.BlockSpec((F, H), lambda i: (0, 0)),
            pl.BlockSpec((H, F), lambda i: (0, 0)),
            pl.BlockSpec((F, B), lambda i: (0, 0)),
            pl.BlockSpec((1, H), lambda i: (0, 0)),
            pl.BlockSpec((H, B), lambda i: (0, 0)),
        ],
        out_specs=pl.BlockSpec((1, B), lambda i: (0, 0)),
        scratch_shapes=[
            pltpu.VMEM((H, B), jnp.float32),
            pltpu.VMEM((1, B), jnp.float32),
        ],
        out_shape=jax.ShapeDtypeStruct((1, B), jnp.int32),
        compiler_params=pltpu.CompilerParams(
            dimension_semantics=("arbitrary",),
        ),
    )(memory, memory, wT, hash_W, flatT, thr_row, thr_col)
    return idx2d.reshape(B)


@functools.cache
def _make_sc_gather():
    mesh = plsc.VectorSubcoreMesh(core_axis_name="c", subcore_axis_name="s")
    rows_per_worker = 8
    n_workers = B // rows_per_worker

    @functools.partial(
        pl.kernel,
        mesh=mesh,
        out_type=jax.ShapeDtypeStruct((B, F), jnp.float32),
        scratch_types=[
            pltpu.VMEM((rows_per_worker,), jnp.int32),
            pltpu.VMEM((rows_per_worker, F), jnp.float32),
            pltpu.SemaphoreType.DMA,
        ],
    )
    def gather_k(table_hbm, idx_hbm, out_hbm, idx_v, rows_v, sem):
        wid = lax.axis_index("s") * 2 + lax.axis_index("c")

        @pl.when(wid < n_workers)
        def _():
            base = wid * rows_per_worker
            pltpu.sync_copy(idx_hbm.at[pl.ds(base, rows_per_worker)], idx_v)
            pltpu.async_copy(table_hbm.at[idx_v], rows_v, sem).wait()
            pltpu.sync_copy(rows_v, out_hbm.at[pl.ds(base, rows_per_worker)])

    return gather_k


def kernel(feature, memory, hash_W, hash_b):
    b, c, h, w = feature.shape
    flat = feature.reshape(b, c * h * w)
    idx = _hash_argmin(memory, flat, hash_W, hash_b)
    recon = _make_sc_gather()(memory, idx)
    return recon.reshape(b, c, h, w)


# speedup vs baseline: 1.0337x; 1.0337x over previous
import functools

import jax
import jax.numpy as jnp
from jax import lax
from jax.experimental import pallas as pl
from jax.experimental.pallas import tpu as pltpu
from jax.experimental.pallas import tpu_sc as plsc

M = 8192
F = 4096
H = 128
B = 32
MB = 512
NBLK = M // MB


def _argmin_body(mem_ref, wT_ref, W_ref, flatT_ref, thr_row_ref, thr_col_ref,
                 out_ref, gT_ref, min_ref):
    i = pl.program_id(0)

    @pl.when(i == 0)
    def _init():
        hfT_logits = jnp.dot(W_ref[...], flatT_ref[...],
                             preferred_element_type=jnp.float32)
        hfT = (hfT_logits > thr_col_ref[...]).astype(jnp.float32)
        gT_ref[...] = 1.0 - 2.0 * hfT
        min_ref[...] = jnp.full((1, B), 1e30, jnp.float32)

    logits = jnp.dot(mem_ref[...], wT_ref[...],
                     preferred_element_type=jnp.float32)
    hm = (logits > thr_row_ref[...]).astype(jnp.float32)
    sT = jnp.dot(hm, gT_ref[...], preferred_element_type=jnp.float32)
    m_ids = lax.broadcasted_iota(jnp.int32, (MB, B), 0).astype(jnp.float32)
    key = sT * float(M) + (m_ids + jnp.float32(i * MB))
    min_ref[...] = jnp.minimum(min_ref[...],
                               jnp.min(key, axis=0, keepdims=True))

    @pl.when(i == NBLK - 1)
    def _fin():
        k = min_ref[...]
        idx = k - jnp.floor(k / float(M)) * float(M)
        out_ref[...] = idx.astype(jnp.int32)


def _hash_argmin(memory, flat, hash_W, hash_b):
    wT = hash_W.T
    flatT = flat.T
    thr = 0.5 - hash_b
    thr_row = thr[None, :]
    thr_col = jnp.broadcast_to(thr[:, None], (H, B))

    idx2d = pl.pallas_call(
        _argmin_body,
        grid=(NBLK,),
        in_specs=[
            pl.BlockSpec((MB, F), lambda i: (i, 0)),
            pl.BlockSpec((F, H), lambda i: (0, 0)),
            pl.BlockSpec((H, F), lambda i: (0, 0)),
            pl.BlockSpec((F, B), lambda i: (0, 0)),
            pl.BlockSpec((1, H), lambda i: (0, 0)),
            pl.BlockSpec((H, B), lambda i: (0, 0)),
        ],
        out_specs=pl.BlockSpec((1, B), lambda i: (0, 0)),
        scratch_shapes=[
            pltpu.VMEM((H, B), jnp.float32),
            pltpu.VMEM((1, B), jnp.float32),
        ],
        out_shape=jax.ShapeDtypeStruct((1, B), jnp.int32),
        compiler_params=pltpu.CompilerParams(
            dimension_semantics=("arbitrary",),
        ),
    )(memory, wT, hash_W, flatT, thr_row, thr_col)
    return idx2d.reshape(B)


@functools.cache
def _make_sc_gather():
    mesh = plsc.VectorSubcoreMesh(core_axis_name="c", subcore_axis_name="s",
                                  num_cores=1)
    rows_per_worker = 8
    n_workers = B // rows_per_worker

    @functools.partial(
        pl.kernel,
        mesh=mesh,
        out_type=jax.ShapeDtypeStruct((B, F), jnp.float32),
        scratch_types=[
            pltpu.VMEM((rows_per_worker,), jnp.int32),
            pltpu.VMEM((rows_per_worker, F), jnp.float32),
            pltpu.SemaphoreType.DMA,
        ],
    )
    def gather_k(table_hbm, idx_hbm, out_hbm, idx_v, rows_v, sem):
        wid = lax.axis_index("s")

        @pl.when(wid < n_workers)
        def _():
            base = wid * rows_per_worker
            pltpu.sync_copy(idx_hbm.at[pl.ds(base, rows_per_worker)], idx_v)
            pltpu.async_copy(table_hbm.at[idx_v], rows_v, sem).wait()
            pltpu.sync_copy(rows_v, out_hbm.at[pl.ds(base, rows_per_worker)])

    return gather_k


def kernel(feature, memory, hash_W, hash_b):
    b, c, h, w = feature.shape
    flat = feature.reshape(b, c * h * w)
    idx = _hash_argmin(memory, flat, hash_W, hash_b)
    recon = _make_sc_gather()(memory, idx)
    return recon.reshape(b, c, h, w)


# speedup vs baseline: 1.0793x; 1.0440x over previous
import functools

import jax
import jax.numpy as jnp
from jax import lax
from jax.experimental import pallas as pl
from jax.experimental.pallas import tpu as pltpu
from jax.experimental.pallas import tpu_sc as plsc

M = 8192
F = 4096
H = 128
B = 32
MB = 512
NBLK = M // MB

_DN_T = (((1,), (1,)), ((), ()))


def _dot_t(a, b):
    return lax.dot_general(a, b, _DN_T, preferred_element_type=jnp.float32)


def _argmin_body(mem_ref, W_ref, flat_ref, thr_ref, out_ref, g_ref, min_ref):
    i = pl.program_id(0)

    @pl.when(i == 0)
    def _init():
        hf_logits = _dot_t(flat_ref[...], W_ref[...])
        hf = (hf_logits > thr_ref[...]).astype(jnp.float32)
        g_ref[...] = 1.0 - 2.0 * hf
        min_ref[...] = jnp.full((1, B), 1e30, jnp.float32)

    logits = _dot_t(mem_ref[...], W_ref[...])
    hm = (logits > thr_ref[...]).astype(jnp.float32)
    sT = _dot_t(hm, g_ref[...])
    m_ids = lax.broadcasted_iota(jnp.int32, (MB, B), 0).astype(jnp.float32)
    key = sT * float(M) + (m_ids + jnp.float32(i * MB))
    min_ref[...] = jnp.minimum(min_ref[...],
                               jnp.min(key, axis=0, keepdims=True))

    @pl.when(i == NBLK - 1)
    def _fin():
        k = min_ref[...]
        idx = k - jnp.floor(k / float(M)) * float(M)
        out_ref[...] = idx.astype(jnp.int32)


def _hash_argmin(memory, flat, hash_W, hash_b):
    thr = (0.5 - hash_b)[None, :]

    idx2d = pl.pallas_call(
        _argmin_body,
        grid=(NBLK,),
        in_specs=[
            pl.BlockSpec((MB, F), lambda i: (i, 0)),
            pl.BlockSpec((H, F), lambda i: (0, 0)),
            pl.BlockSpec((B, F), lambda i: (0, 0)),
            pl.BlockSpec((1, H), lambda i: (0, 0)),
        ],
        out_specs=pl.BlockSpec((1, B), lambda i: (0, 0)),
        scratch_shapes=[
            pltpu.VMEM((B, H), jnp.float32),
            pltpu.VMEM((1, B), jnp.float32),
        ],
        out_shape=jax.ShapeDtypeStruct((1, B), jnp.int32),
        compiler_params=pltpu.CompilerParams(
            dimension_semantics=("arbitrary",),
        ),
    )(memory, hash_W, flat, thr)
    return idx2d.reshape(B)


@functools.cache
def _make_sc_gather():
    mesh = plsc.VectorSubcoreMesh(core_axis_name="c", subcore_axis_name="s",
                                  num_cores=1)
    rows_per_worker = 8
    n_workers = B // rows_per_worker

    @functools.partial(
        pl.kernel,
        mesh=mesh,
        out_type=jax.ShapeDtypeStruct((B, F), jnp.float32),
        scratch_types=[
            pltpu.VMEM((rows_per_worker,), jnp.int32),
            pltpu.VMEM((rows_per_worker, F), jnp.float32),
            pltpu.SemaphoreType.DMA,
        ],
    )
    def gather_k(table_hbm, idx_hbm, out_hbm, idx_v, rows_v, sem):
        wid = lax.axis_index("s")

        @pl.when(wid < n_workers)
        def _():
            base = wid * rows_per_worker
            pltpu.sync_copy(idx_hbm.at[pl.ds(base, rows_per_worker)], idx_v)
            pltpu.async_copy(table_hbm.at[idx_v], rows_v, sem).wait()
            pltpu.sync_copy(rows_v, out_hbm.at[pl.ds(base, rows_per_worker)])

    return gather_k


def kernel(feature, memory, hash_W, hash_b):
    b, c, h, w = feature.shape
    flat = feature.reshape(b, c * h * w)
    idx = _hash_argmin(memory, flat, hash_W, hash_b)
    recon = _make_sc_gather()(memory, idx)
    return recon.reshape(b, c, h, w)


# speedup vs baseline: 1.0892x; 1.0093x over previous
import functools

import jax
import jax.numpy as jnp
from jax import lax
from jax.experimental import pallas as pl
from jax.experimental.pallas import tpu as pltpu
from jax.experimental.pallas import tpu_sc as plsc

M = 8192
F = 4096
H = 128
B = 32
MB = 512
NBLK = M // MB

_DN_T = (((1,), (1,)), ((), ()))


def _dot_t(a, b):
    return lax.dot_general(a, b, _DN_T, preferred_element_type=jnp.float32)


def _argmin_body(mem_ref, W_ref, flat_ref, b_ref, out_ref, g_ref, min_ref):
    i = pl.program_id(0)
    thr = 0.5 - b_ref[...]

    @pl.when(i == 0)
    def _init():
        hf_logits = _dot_t(flat_ref[...], W_ref[...])
        hf = (hf_logits > thr).astype(jnp.float32)
        g_ref[...] = 1.0 - 2.0 * hf
        min_ref[...] = jnp.full((1, B), 1e30, jnp.float32)

    logits = _dot_t(mem_ref[...], W_ref[...])
    hm = (logits > thr).astype(jnp.float32)
    sT = _dot_t(hm, g_ref[...])
    m_ids = lax.broadcasted_iota(jnp.int32, (MB, B), 0).astype(jnp.float32)
    key = sT * float(M) + (m_ids + jnp.float32(i * MB))
    min_ref[...] = jnp.minimum(min_ref[...],
                               jnp.min(key, axis=0, keepdims=True))

    @pl.when(i == NBLK - 1)
    def _fin():
        k = min_ref[...]
        idx = k - jnp.floor(k / float(M)) * float(M)
        out_ref[...] = idx.astype(jnp.int32)


def _hash_argmin(memory, flat, hash_W, hash_b):
    idx2d = pl.pallas_call(
        _argmin_body,
        grid=(NBLK,),
        in_specs=[
            pl.BlockSpec((MB, F), lambda i: (i, 0)),
            pl.BlockSpec((H, F), lambda i: (0, 0)),
            pl.BlockSpec((B, F), lambda i: (0, 0)),
            pl.BlockSpec((1, H), lambda i: (0, 0)),
        ],
        out_specs=pl.BlockSpec((1, B), lambda i: (0, 0)),
        scratch_shapes=[
            pltpu.VMEM((B, H), jnp.float32),
            pltpu.VMEM((1, B), jnp.float32),
        ],
        out_shape=jax.ShapeDtypeStruct((1, B), jnp.int32),
        compiler_params=pltpu.CompilerParams(
            dimension_semantics=("arbitrary",),
        ),
    )(memory, hash_W, flat, hash_b.reshape(1, H))
    return idx2d.reshape(B)


@functools.cache
def _make_sc_gather():
    mesh = plsc.VectorSubcoreMesh(core_axis_name="c", subcore_axis_name="s",
                                  num_cores=1)
    rows_per_worker = 8
    n_workers = B // rows_per_worker

    @functools.partial(
        pl.kernel,
        mesh=mesh,
        out_type=jax.ShapeDtypeStruct((B, F), jnp.float32),
        scratch_types=[
            pltpu.VMEM((rows_per_worker,), jnp.int32),
            pltpu.VMEM((rows_per_worker, F), jnp.float32),
            pltpu.SemaphoreType.DMA,
        ],
    )
    def gather_k(table_hbm, idx_hbm, out_hbm, idx_v, rows_v, sem):
        wid = lax.axis_index("s")

        @pl.when(wid < n_workers)
        def _():
            base = wid * rows_per_worker
            pltpu.sync_copy(idx_hbm.at[pl.ds(base, rows_per_worker)], idx_v)
            pltpu.async_copy(table_hbm.at[idx_v], rows_v, sem).wait()
            pltpu.sync_copy(rows_v, out_hbm.at[pl.ds(base, rows_per_worker)])

    return gather_k


def kernel(feature, memory, hash_W, hash_b):
    b, c, h, w = feature.shape
    flat = feature.reshape(b, c * h * w)
    idx = _hash_argmin(memory, flat, hash_W, hash_b)
    recon = _make_sc_gather()(memory, idx)
    return recon.reshape(b, c, h, w)
